# baseline (speedup 1.0000x reference)
import functools

import jax
import jax.numpy as jnp
from jax import lax
from jax.experimental import pallas as pl
from jax.experimental.pallas import tpu as pltpu
from jax.experimental.pallas import tpu_sc as plsc

D_MODEL = 64
SCALE = 8.0

_NC = 2
_NS = 16
_NW = _NC * _NS
_G = 128
_NBUF = 10
_NOBUF = 2
_OSTRIDE = _G + 1


@functools.lru_cache(maxsize=None)
def _make_kernel(n_t, n_bb):
    per_w = (n_t * n_bb) // _NW
    mesh = plsc.VectorSubcoreMesh(core_axis_name="c", subcore_axis_name="s")

    @functools.partial(
        pl.kernel,
        mesh=mesh,
        out_type=jax.ShapeDtypeStruct((n_t, 8, n_bb, 8, _G), jnp.float32),
        scratch_types=(
            [pltpu.VMEM((per_w, _G), jnp.int32)]
            + [pltpu.VMEM((_G, D_MODEL), jnp.float32) for _ in range(_NBUF)]
            + [pltpu.VMEM((8, 8, _OSTRIDE), jnp.float32)
               for _ in range(_NOBUF)]
            + [pltpu.SemaphoreType.DMA for _ in range(_NBUF + _NOBUF)]
        ),
        compiler_params=pltpu.CompilerParams(
            use_tc_tiling_on_sc=False, needs_layout_passes=False),
    )
    def emb(x_hbm, table_hbm, out_hbm, idx_v, *rest):
        rows = rest[:_NBUF]
        obuf = rest[_NBUF:_NBUF + _NOBUF]
        gsem = rest[_NBUF + _NOBUF:2 * _NBUF + _NOBUF]
        osem = rest[2 * _NBUF + _NOBUF:]

        wid = lax.axis_index("s") * _NC + lax.axis_index("c")
        g0 = wid * per_w
        pltpu.sync_copy(x_hbm.at[wid], idx_v)

        iota16 = lax.iota(jnp.int32, 16)
        cq = [(iota16 + 16 * q) // 8 for q in range(D_MODEL // 16)]
        sq = [(iota16 + 16 * q) % 8 for q in range(D_MODEL // 16)]

        def start_gather(g, b):
            pltpu.async_copy(table_hbm.at[idx_v.at[g]], rows[b], gsem[b])

        def wait_gather(g, b):
            pltpu.make_async_copy(
                table_hbm.at[idx_v.at[g]], rows[b], gsem[b]).wait()

        def start_out(g, o):
            G = g0 + g
            t = G // n_bb
            bb = G % n_bb
            pltpu.async_copy(
                obuf[o].at[:, :, pl.ds(0, _G)], out_hbm.at[t, :, bb],
                osem[o])

        def wait_out(o):
            pltpu.make_async_copy(
                obuf[o].at[:, :, pl.ds(0, _G)], out_hbm.at[0, :, 0],
                osem[o]).wait()

        def transpose_scale(b, o):
            src = rows[b]
            dst = obuf[o]

            @plsc.parallel_loop(0, _G, unroll=4)
            def _(l):
                lv = jnp.full((16,), 0, jnp.int32) + l
                for q in range(D_MODEL // 16):
                    v = src[l, pl.ds(16 * q, 16)]
                    plsc.store_scatter(dst, [cq[q], sq[q], lv], v * SCALE)

        for b in range(_NBUF):
            start_gather(b, b)

        for j in range(_NBUF):
            o = j % _NOBUF
            wait_gather(j, j)
            if j >= _NOBUF:
                wait_out(o)
            transpose_scale(j, o)
            start_gather(j + _NBUF, j)
            start_out(j, o)

        n_blocks = per_w // _NBUF

        def block(blk, carry):
            for j in range(_NBUF):
                g = blk * _NBUF + j
                o = j % _NOBUF
                wait_gather(g, j)
                wait_out(o)
                transpose_scale(j, o)
                ng = g + _NBUF

                @pl.when(ng < per_w)
                def _():
                    start_gather(ng, j)

                start_out(g, o)
            return carry

        lax.fori_loop(1, n_blocks, block, 0)

        for o in range(_NOBUF):
            wait_out(o)

    return emb


def kernel(x, table):
    bsz, seq = x.shape
    n_bb = bsz // _G
    x3 = (2 * x.T.astype(jnp.int32)).reshape(
        _NW, (bsz * seq) // (_NW * _G), _G)
    tpad = jnp.pad(table, ((0, 0), (0, 2 * D_MODEL - table.shape[1])))
    t2 = tpad.reshape(2 * table.shape[0], D_MODEL)
    buf = _make_kernel(seq, n_bb)(x3, t2)
    return buf.transpose(2, 4, 0, 1, 3).reshape(bsz, seq, D_MODEL)


# speedup vs baseline: 1.0033x; 1.0033x over previous
import functools

import jax
import jax.numpy as jnp
from jax import lax
from jax.experimental import pallas as pl
from jax.experimental.pallas import tpu as pltpu
from jax.experimental.pallas import tpu_sc as plsc

D_MODEL = 64
SCALE = 8.0

_NC = 2
_NS = 16
_NW = _NC * _NS
_G = 128
_NBUF = 8
_NOBUF = 4
_OSTRIDE = _G + 1


@functools.lru_cache(maxsize=None)
def _make_kernel(n_t, n_bb):
    per_w = (n_t * n_bb) // _NW
    mesh = plsc.VectorSubcoreMesh(core_axis_name="c", subcore_axis_name="s")

    @functools.partial(
        pl.kernel,
        mesh=mesh,
        out_type=jax.ShapeDtypeStruct((n_t, 8, n_bb, 8, _G), jnp.float32),
        scratch_types=(
            [pltpu.VMEM((per_w, _G), jnp.int32)]
            + [pltpu.VMEM((_G, D_MODEL), jnp.float32) for _ in range(_NBUF)]
            + [pltpu.VMEM((8, 8, _OSTRIDE), jnp.float32)
               for _ in range(_NOBUF)]
            + [pltpu.SemaphoreType.DMA for _ in range(_NBUF + _NOBUF)]
        ),
        compiler_params=pltpu.CompilerParams(
            use_tc_tiling_on_sc=False, needs_layout_passes=False),
    )
    def emb(x_hbm, table_hbm, out_hbm, idx_v, *rest):
        rows = rest[:_NBUF]
        obuf = rest[_NBUF:_NBUF + _NOBUF]
        gsem = rest[_NBUF + _NOBUF:2 * _NBUF + _NOBUF]
        osem = rest[2 * _NBUF + _NOBUF:]

        wid = lax.axis_index("s") * _NC + lax.axis_index("c")
        g0 = wid * per_w
        pltpu.sync_copy(x_hbm.at[wid], idx_v)

        iota16 = lax.iota(jnp.int32, 16)
        cq = [(iota16 + 16 * q) // 8 for q in range(D_MODEL // 16)]
        sq = [(iota16 + 16 * q) % 8 for q in range(D_MODEL // 16)]

        def start_gather(g, b):
            pltpu.async_copy(table_hbm.at[idx_v.at[g]], rows[b], gsem[b])

        def wait_gather(g, b):
            pltpu.make_async_copy(
                table_hbm.at[idx_v.at[g]], rows[b], gsem[b]).wait()

        def start_out(g, o):
            G = g0 + g
            t = G // n_bb
            bb = G % n_bb
            pltpu.async_copy(
                obuf[o].at[:, :, pl.ds(0, _G)], out_hbm.at[t, :, bb],
                osem[o])

        def wait_out(o):
            pltpu.make_async_copy(
                obuf[o].at[:, :, pl.ds(0, _G)], out_hbm.at[0, :, 0],
                osem[o]).wait()

        def transpose_scale(b, o):
            src = rows[b]
            dst = obuf[o]

            @plsc.parallel_loop(0, _G, unroll=4)
            def _(l):
                lv = jnp.full((16,), 0, jnp.int32) + l
                for q in range(D_MODEL // 16):
                    v = src[l, pl.ds(16 * q, 16)]
                    plsc.store_scatter(dst, [cq[q], sq[q], lv], v * SCALE)

        for b in range(_NBUF):
            start_gather(b, b)

        for j in range(_NBUF):
            o = j % _NOBUF
            wait_gather(j, j)
            if j >= _NOBUF:
                wait_out(o)
            transpose_scale(j, o)
            start_gather(j + _NBUF, j)
            start_out(j, o)

        n_blocks = per_w // _NBUF

        def block(blk, carry):
            for j in range(_NBUF):
                g = blk * _NBUF + j
                o = j % _NOBUF
                wait_gather(g, j)
                wait_out(o)
                transpose_scale(j, o)
                ng = g + _NBUF

                @pl.when(ng < per_w)
                def _():
                    start_gather(ng, j)

                start_out(g, o)
            return carry

        lax.fori_loop(1, n_blocks, block, 0)

        for o in range(_NOBUF):
            wait_out(o)

    return emb


def kernel(x, table):
    bsz, seq = x.shape
    n_bb = bsz // _G
    x3 = (2 * x.T.astype(jnp.int32)).reshape(
        _NW, (bsz * seq) // (_NW * _G), _G)
    tpad = jnp.pad(table, ((0, 0), (0, 2 * D_MODEL - table.shape[1])))
    t2 = tpad.reshape(2 * table.shape[0], D_MODEL)
    buf = _make_kernel(seq, n_bb)(x3, t2)
    return buf.transpose(2, 4, 0, 1, 3).reshape(bsz, seq, D_MODEL)
